# baseline (speedup 1.0000x reference)
import functools

import jax
import jax.numpy as jnp
from jax.experimental import pallas as pl
from jax.experimental.pallas import tpu as pltpu

EPS = 1e-6
CAP = 0.5


def _rmsnorm(x, w):
    var = jnp.mean(x * x, axis=-1, keepdims=True)
    return w * (x * jax.lax.rsqrt(var + EPS))


def _dot16(a, b):
    return jnp.dot(a.astype(jnp.bfloat16), b, preferred_element_type=jnp.float32)


def _mid_kernel(x_ref, nw_ref, wg_ref, wu_ref, mid_ref, xn_ref):
    j = pl.program_id(1)

    @pl.when(j == 0)
    def _():
        xn_ref[...] = _rmsnorm(x_ref[...], nw_ref[...]).astype(jnp.bfloat16)

    xn = xn_ref[...]
    g = jnp.dot(xn, wg_ref[...], preferred_element_type=jnp.float32)
    u = jnp.dot(xn, wu_ref[...], preferred_element_type=jnp.float32)
    mid_ref[...] = (jax.nn.silu(g) * u).astype(jnp.bfloat16)


def _mid_mlp(x, norm_w, wg, wu, tok, fb):
    n, d = x.shape
    f = wg.shape[1]
    nt, nf = n // tok, f // fb
    return pl.pallas_call(
        _mid_kernel,
        grid=(nt, nf),
        in_specs=[
            pl.BlockSpec((tok, d), lambda i, j: (i, 0)),
            pl.BlockSpec((1, d), lambda i, j: (0, 0)),
            pl.BlockSpec((d, fb), lambda i, j: (0, j)),
            pl.BlockSpec((d, fb), lambda i, j: (0, j)),
        ],
        out_specs=pl.BlockSpec((tok, fb), lambda i, j: (i, j)),
        out_shape=jax.ShapeDtypeStruct((n, f), jnp.bfloat16),
        scratch_shapes=[pltpu.VMEM((tok, d), jnp.bfloat16)],
        compiler_params=pltpu.CompilerParams(
            dimension_semantics=("parallel", "arbitrary")),
    )(x, norm_w, wg, wu)


def _down_kernel(mid_ref, wd_ref, h_ref):
    h_ref[...] = jnp.dot(mid_ref[...], wd_ref[...],
                         preferred_element_type=jnp.float32)


def _down_mlp(mid, wd, tok, db):
    n, f = mid.shape
    d = wd.shape[1]
    nt, nd = n // tok, d // db
    return pl.pallas_call(
        _down_kernel,
        grid=(nd, nt),
        in_specs=[
            pl.BlockSpec((tok, f), lambda j, i: (i, 0)),
            pl.BlockSpec((f, db), lambda j, i: (0, j)),
        ],
        out_specs=pl.BlockSpec((tok, db), lambda j, i: (i, j)),
        out_shape=jax.ShapeDtypeStruct((n, d), jnp.float32),
        compiler_params=pltpu.CompilerParams(
            dimension_semantics=("parallel", "parallel")),
    )(mid, wd)


def _tpn_kernel(px_ref, ph_ref, h_ref, x_ref, tnw_ref, twg_ref, twu_ref,
                twd_ref, rw_ref, bce_ref, bcu_ref,
                g_ref, lg_ref, dch_ref, *, d):
    pn = _rmsnorm(px_ref[...] + ph_ref[...], tnw_ref[...]).astype(jnp.bfloat16)
    tg = jnp.dot(pn, twg_ref[...], preferred_element_type=jnp.float32)
    tu = jnp.dot(pn, twu_ref[...], preferred_element_type=jnp.float32)
    pred = _dot16(jax.nn.silu(tg) * tu, twd_ref[...])
    hh = h_ref[...]
    dst = jnp.sum(hh * hh, axis=1, keepdims=True) / d
    diff = pred - hh
    dch = jnp.sum(diff * diff, axis=1, keepdims=True) / d
    g_ref[...] = jax.nn.sigmoid(bce_ref[0, 0] * dst - bcu_ref[0, 0] * dch)
    lg_ref[...] = jnp.sum(x_ref[...] * rw_ref[...], axis=1, keepdims=True)
    dch_ref[...] = dch


def _tpn(px, ph, h, x, tnw, twg, twu, twd, rw, bce, bcu, tok):
    n, d = x.shape
    f = twg.shape[1]
    nt = n // tok
    outs = [jax.ShapeDtypeStruct((n, 1), jnp.float32)] * 3
    return pl.pallas_call(
        functools.partial(_tpn_kernel, d=float(d)),
        grid=(nt,),
        in_specs=[
            pl.BlockSpec((tok, d), lambda i: (i, 0)),
            pl.BlockSpec((tok, d), lambda i: (i, 0)),
            pl.BlockSpec((tok, d), lambda i: (i, 0)),
            pl.BlockSpec((tok, d), lambda i: (i, 0)),
            pl.BlockSpec((1, d), lambda i: (0, 0)),
            pl.BlockSpec((d, f), lambda i: (0, 0)),
            pl.BlockSpec((d, f), lambda i: (0, 0)),
            pl.BlockSpec((f, d), lambda i: (0, 0)),
            pl.BlockSpec((1, d), lambda i: (0, 0)),
            pl.BlockSpec(memory_space=pltpu.SMEM),
            pl.BlockSpec(memory_space=pltpu.SMEM),
        ],
        out_specs=[pl.BlockSpec((tok, 1), lambda i: (i, 0))] * 3,
        out_shape=outs,
        compiler_params=pltpu.CompilerParams(
            dimension_semantics=("arbitrary",)),
    )(px, ph, h, x, tnw, twg, twu, twd, rw, bce, bcu)


def _select_kernel(g_ref, lg_ref, dch_ref, gm_ref, tl_ref, cl_ref, *, k, tchunk):
    b, t = g_ref.shape
    g = g_ref[...]
    nc = t // tchunk
    kf = jnp.float32(k)
    for c in range(nc):
        gt = g[:, c * tchunk:(c + 1) * tchunk]
        gt_b = gt[:, None, :]
        it = jax.lax.broadcasted_iota(jnp.int32, (1, 1, tchunk), 2) + c * tchunk
        cnt = jnp.zeros((b, tchunk), jnp.float32)
        for cp in range(nc):
            gp = g[:, cp * tchunk:(cp + 1) * tchunk]
            gp_b = gp[:, :, None]
            ip = jax.lax.broadcasted_iota(jnp.int32, (1, tchunk, 1), 1) + cp * tchunk
            beat = (gp_b > gt_b) | ((gp_b == gt_b) & (ip < it))
            cnt = cnt + jnp.sum(beat.astype(jnp.float32), axis=1)
        sel = cnt < kf
        gm_ref[:, c * tchunk:(c + 1) * tchunk] = jnp.where(sel, gt, 0.0)
    tl_ref[0, 0] = jnp.sum(dch_ref[...]) / (b * t)
    lg = lg_ref[...]
    tgt = (gm_ref[...] > 0.0).astype(jnp.float32)
    bce = jnp.maximum(lg, 0.0) - lg * tgt + jnp.log1p(jnp.exp(-jnp.abs(lg)))
    cl_ref[0, 0] = jnp.sum(bce) / (b * t)


def _select(g, lg, dch, k, tchunk):
    b, t = g.shape
    return pl.pallas_call(
        functools.partial(_select_kernel, k=k, tchunk=tchunk),
        in_specs=[pl.BlockSpec((b, t), lambda: (0, 0))] * 3,
        out_specs=[pl.BlockSpec((b, t), lambda: (0, 0)),
                   pl.BlockSpec(memory_space=pltpu.SMEM),
                   pl.BlockSpec(memory_space=pltpu.SMEM)],
        out_shape=[jax.ShapeDtypeStruct((b, t), jnp.float32),
                   jax.ShapeDtypeStruct((1, 1), jnp.float32),
                   jax.ShapeDtypeStruct((1, 1), jnp.float32)],
    )(g, lg, dch)


def _combine_kernel(x_ref, h_ref, gm_ref, o_ref):
    o_ref[...] = x_ref[...] + gm_ref[...] * h_ref[...]


def _combine(x, h, gm, tok):
    n, d = x.shape
    return pl.pallas_call(
        _combine_kernel,
        grid=(n // tok,),
        in_specs=[
            pl.BlockSpec((tok, d), lambda i: (i, 0)),
            pl.BlockSpec((tok, d), lambda i: (i, 0)),
            pl.BlockSpec((tok, 1), lambda i: (i, 0)),
        ],
        out_specs=pl.BlockSpec((tok, d), lambda i: (i, 0)),
        out_shape=jax.ShapeDtypeStruct((n, d), jnp.float32),
        compiler_params=pltpu.CompilerParams(
            dimension_semantics=("parallel",)),
    )(x, h, gm)


def kernel(hidden_states, beta_ce, beta_cu, norm_w, wg, wu, wd,
           tpn_norm_w, tpn_wg, tpn_wu, tpn_wd, router_w):
    b, t, d = hidden_states.shape
    f = wg.shape[1]
    ft = tpn_wg.shape[1]
    n = b * t
    k = max(1, int(t * CAP))

    tok_a = min(1024, n)
    fb = min(512, f)
    tok_a2 = min(512, n)
    db = min(512, d)
    tok_b = min(256, n)
    tchunk = min(512, t)

    x = hidden_states.reshape(n, d)
    nw = norm_w.reshape(1, d)
    tnw = tpn_norm_w.reshape(1, d)
    rw = router_w.reshape(1, d)
    bce = beta_ce.reshape(1, 1)
    bcu = beta_cu.reshape(1, 1)

    bf = jnp.bfloat16
    wg16, wu16, wd16 = wg.astype(bf), wu.astype(bf), wd.astype(bf)
    twg16, twu16, twd16 = tpn_wg.astype(bf), tpn_wu.astype(bf), tpn_wd.astype(bf)

    mid = _mid_mlp(x, nw, wg16, wu16, tok_a, fb)
    h = _down_mlp(mid, wd16, tok_a2, db)

    hs3 = hidden_states
    h3 = h.reshape(b, t, d)
    zero = jnp.zeros((b, 1, d), jnp.float32)
    px = jnp.concatenate([zero, hs3[:, :-1]], axis=1).reshape(n, d)
    ph = jnp.concatenate([zero, h3[:, :-1]], axis=1).reshape(n, d)

    g, lg, dch = _tpn(px, ph, h, x, tnw, twg16, twu16, twd16, rw, bce, bcu,
                      tok_b)

    gm, tl, cl = _select(g.reshape(b, t), lg.reshape(b, t), dch.reshape(b, t),
                         k, tchunk)

    out = _combine(x, h, gm.reshape(n, 1), tok_b)

    return (out.reshape(b, t, d), g.reshape(b, t), tl.reshape(()),
            cl.reshape(()))


# speedup vs baseline: 1.0694x; 1.0694x over previous
import functools

import jax
import jax.numpy as jnp
from jax.experimental import pallas as pl
from jax.experimental.pallas import tpu as pltpu

EPS = 1e-6
CAP = 0.5


def _rmsnorm(x, w):
    var = jnp.mean(x * x, axis=-1, keepdims=True)
    return w * (x * jax.lax.rsqrt(var + EPS))


def _dot16(a, b):
    return jnp.dot(a.astype(jnp.bfloat16), b, preferred_element_type=jnp.float32)


def _mid_kernel(x_ref, nw_ref, wg_ref, wu_ref, mid_ref, xn_ref):
    j = pl.program_id(1)

    @pl.when(j == 0)
    def _():
        xn_ref[...] = _rmsnorm(x_ref[...], nw_ref[...]).astype(jnp.bfloat16)

    xn = xn_ref[...]
    g = jnp.dot(xn, wg_ref[...], preferred_element_type=jnp.float32)
    u = jnp.dot(xn, wu_ref[...], preferred_element_type=jnp.float32)
    mid_ref[...] = (jax.nn.silu(g) * u).astype(jnp.bfloat16)


def _mid_mlp(x, norm_w, wg, wu, tok, fb):
    n, d = x.shape
    f = wg.shape[1]
    nt, nf = n // tok, f // fb
    return pl.pallas_call(
        _mid_kernel,
        grid=(nt, nf),
        in_specs=[
            pl.BlockSpec((tok, d), lambda i, j: (i, 0)),
            pl.BlockSpec((1, d), lambda i, j: (0, 0)),
            pl.BlockSpec((d, fb), lambda i, j: (0, j)),
            pl.BlockSpec((d, fb), lambda i, j: (0, j)),
        ],
        out_specs=pl.BlockSpec((tok, fb), lambda i, j: (i, j)),
        out_shape=jax.ShapeDtypeStruct((n, f), jnp.bfloat16),
        scratch_shapes=[pltpu.VMEM((tok, d), jnp.bfloat16)],
        compiler_params=pltpu.CompilerParams(
            dimension_semantics=("parallel", "arbitrary")),
    )(x, norm_w, wg, wu)


def _down_kernel(mid_ref, wd_ref, h_ref):
    h_ref[...] = jnp.dot(mid_ref[...], wd_ref[...],
                         preferred_element_type=jnp.float32)


def _down_mlp(mid, wd, tok, db):
    n, f = mid.shape
    d = wd.shape[1]
    nt, nd = n // tok, d // db
    return pl.pallas_call(
        _down_kernel,
        grid=(nd, nt),
        in_specs=[
            pl.BlockSpec((tok, f), lambda j, i: (i, 0)),
            pl.BlockSpec((f, db), lambda j, i: (0, j)),
        ],
        out_specs=pl.BlockSpec((tok, db), lambda j, i: (i, j)),
        out_shape=jax.ShapeDtypeStruct((n, d), jnp.float32),
        compiler_params=pltpu.CompilerParams(
            dimension_semantics=("parallel", "parallel")),
    )(mid, wd)


def _tpn_kernel(h_ref, x_ref, tnw_ref, twg_ref, twu_ref,
                twd_ref, rw_ref, bce_ref, bcu_ref,
                g_ref, lg_ref, dch_ref, carry_ref, *, d, tps):
    i = pl.program_id(0)
    tok = x_ref.shape[0]
    hh = h_ref[...]
    s = x_ref[...] + hh
    top = jnp.where((i % tps) == 0, jnp.zeros_like(carry_ref[...]),
                    carry_ref[...])
    carry_ref[...] = s[tok - 1:tok, :]
    prev = jnp.concatenate([top, s[:tok - 1, :]], axis=0)
    pn = _rmsnorm(prev, tnw_ref[...]).astype(jnp.bfloat16)
    tg = jnp.dot(pn, twg_ref[...], preferred_element_type=jnp.float32)
    tu = jnp.dot(pn, twu_ref[...], preferred_element_type=jnp.float32)
    pred = _dot16(jax.nn.silu(tg) * tu, twd_ref[...])
    dst = jnp.sum(hh * hh, axis=1, keepdims=True) / d
    diff = pred - hh
    dch = jnp.sum(diff * diff, axis=1, keepdims=True) / d
    g_ref[...] = jax.nn.sigmoid(bce_ref[0, 0] * dst - bcu_ref[0, 0] * dch)
    lg_ref[...] = jnp.sum(x_ref[...] * rw_ref[...], axis=1, keepdims=True)
    dch_ref[...] = dch


def _tpn(h, x, tnw, twg, twu, twd, rw, bce, bcu, tok, tps):
    n, d = x.shape
    f = twg.shape[1]
    nt = n // tok
    outs = [jax.ShapeDtypeStruct((n, 1), jnp.float32)] * 3
    return pl.pallas_call(
        functools.partial(_tpn_kernel, d=float(d), tps=tps),
        grid=(nt,),
        in_specs=[
            pl.BlockSpec((tok, d), lambda i: (i, 0)),
            pl.BlockSpec((tok, d), lambda i: (i, 0)),
            pl.BlockSpec((1, d), lambda i: (0, 0)),
            pl.BlockSpec((d, f), lambda i: (0, 0)),
            pl.BlockSpec((d, f), lambda i: (0, 0)),
            pl.BlockSpec((f, d), lambda i: (0, 0)),
            pl.BlockSpec((1, d), lambda i: (0, 0)),
            pl.BlockSpec(memory_space=pltpu.SMEM),
            pl.BlockSpec(memory_space=pltpu.SMEM),
        ],
        out_specs=[pl.BlockSpec((tok, 1), lambda i: (i, 0))] * 3,
        out_shape=outs,
        scratch_shapes=[pltpu.VMEM((1, d), jnp.float32)],
        compiler_params=pltpu.CompilerParams(
            dimension_semantics=("arbitrary",)),
    )(h, x, tnw, twg, twu, twd, rw, bce, bcu)


def _select_kernel(g_ref, lg_ref, dch_ref, gm_ref, tl_ref, cl_ref, *, k, tchunk):
    b, t = g_ref.shape
    g = g_ref[...]
    nc = t // tchunk
    kf = jnp.float32(k)
    for c in range(nc):
        gt = g[:, c * tchunk:(c + 1) * tchunk]
        gt_b = gt[:, None, :]
        it = jax.lax.broadcasted_iota(jnp.int32, (1, 1, tchunk), 2) + c * tchunk
        cnt = jnp.zeros((b, tchunk), jnp.float32)
        for cp in range(nc):
            gp = g[:, cp * tchunk:(cp + 1) * tchunk]
            gp_b = gp[:, :, None]
            ip = jax.lax.broadcasted_iota(jnp.int32, (1, tchunk, 1), 1) + cp * tchunk
            beat = (gp_b > gt_b) | ((gp_b == gt_b) & (ip < it))
            cnt = cnt + jnp.sum(beat.astype(jnp.float32), axis=1)
        sel = cnt < kf
        gm_ref[:, c * tchunk:(c + 1) * tchunk] = jnp.where(sel, gt, 0.0)
    tl_ref[0, 0] = jnp.sum(dch_ref[...]) / (b * t)
    lg = lg_ref[...]
    tgt = (gm_ref[...] > 0.0).astype(jnp.float32)
    bce = jnp.maximum(lg, 0.0) - lg * tgt + jnp.log1p(jnp.exp(-jnp.abs(lg)))
    cl_ref[0, 0] = jnp.sum(bce) / (b * t)


def _select(g, lg, dch, k, tchunk):
    b, t = g.shape
    return pl.pallas_call(
        functools.partial(_select_kernel, k=k, tchunk=tchunk),
        in_specs=[pl.BlockSpec((b, t), lambda: (0, 0))] * 3,
        out_specs=[pl.BlockSpec((b, t), lambda: (0, 0)),
                   pl.BlockSpec(memory_space=pltpu.SMEM),
                   pl.BlockSpec(memory_space=pltpu.SMEM)],
        out_shape=[jax.ShapeDtypeStruct((b, t), jnp.float32),
                   jax.ShapeDtypeStruct((1, 1), jnp.float32),
                   jax.ShapeDtypeStruct((1, 1), jnp.float32)],
    )(g, lg, dch)


def _combine_kernel(x_ref, h_ref, gm_ref, o_ref):
    o_ref[...] = x_ref[...] + gm_ref[...] * h_ref[...]


def _combine(x, h, gm, tok):
    n, d = x.shape
    return pl.pallas_call(
        _combine_kernel,
        grid=(n // tok,),
        in_specs=[
            pl.BlockSpec((tok, d), lambda i: (i, 0)),
            pl.BlockSpec((tok, d), lambda i: (i, 0)),
            pl.BlockSpec((tok, 1), lambda i: (i, 0)),
        ],
        out_specs=pl.BlockSpec((tok, d), lambda i: (i, 0)),
        out_shape=jax.ShapeDtypeStruct((n, d), jnp.float32),
        compiler_params=pltpu.CompilerParams(
            dimension_semantics=("parallel",)),
    )(x, h, gm)


def kernel(hidden_states, beta_ce, beta_cu, norm_w, wg, wu, wd,
           tpn_norm_w, tpn_wg, tpn_wu, tpn_wd, router_w):
    b, t, d = hidden_states.shape
    f = wg.shape[1]
    ft = tpn_wg.shape[1]
    n = b * t
    k = max(1, int(t * CAP))

    tok_a = min(1024, n)
    fb = min(1024, f)
    tok_a2 = min(512, n)
    db = min(512, d)
    tok_b = min(256, t)
    tchunk = min(512, t)

    x = hidden_states.reshape(n, d)
    nw = norm_w.reshape(1, d)
    tnw = tpn_norm_w.reshape(1, d)
    rw = router_w.reshape(1, d)
    bce = beta_ce.reshape(1, 1)
    bcu = beta_cu.reshape(1, 1)

    bf = jnp.bfloat16
    wg16, wu16, wd16 = wg.astype(bf), wu.astype(bf), wd.astype(bf)
    twg16, twu16, twd16 = tpn_wg.astype(bf), tpn_wu.astype(bf), tpn_wd.astype(bf)

    mid = _mid_mlp(x, nw, wg16, wu16, tok_a, fb)
    h = _down_mlp(mid, wd16, tok_a2, db)

    g, lg, dch = _tpn(h, x, tnw, twg16, twu16, twd16, rw, bce, bcu,
                      tok_b, t // tok_b)

    gm, tl, cl = _select(g.reshape(b, t), lg.reshape(b, t), dch.reshape(b, t),
                         k, tchunk)

    out = _combine(x, h, gm.reshape(n, 1), tok_b)

    return (out.reshape(b, t, d), g.reshape(b, t), tl.reshape(()),
            cl.reshape(()))
